# baseline (speedup 1.0000x reference)
import functools

import jax
import jax.numpy as jnp
from jax import lax
from jax.experimental import pallas as pl
from jax.experimental.pallas import tpu as pltpu
from jax.experimental.pallas import tpu_sc as plsc

N = 10000
H = 256
NODE_D = 128

_NC, _NS = 2, 16
_NW = _NC * _NS
_B_PAD = 10240
_BPW = _B_PAD // _NW
_CHUNK = 80

_BM = 200

_DOT = functools.partial(jnp.dot, precision=lax.Precision.DEFAULT,
                         preferred_element_type=jnp.float32)


def _sc_gather_body(idx_hbm, table_hbm, out_hbm, idx_v, rows_v, *sems):
  wid = lax.axis_index("s") * _NC + lax.axis_index("c")
  base = wid * _BPW
  pltpu.sync_copy(idx_hbm.at[pl.ds(base, _BPW)], idx_v)
  nch = _BPW // _CHUNK
  gathers = [
      pltpu.async_copy(
          table_hbm.at[idx_v.at[pl.ds(c * _CHUNK, _CHUNK)]],
          rows_v.at[pl.ds(c * _CHUNK, _CHUNK)],
          sems[c],
      )
      for c in range(nch)
  ]
  writes = []
  for c in range(nch):
    gathers[c].wait()
    writes.append(pltpu.async_copy(
        rows_v.at[pl.ds(c * _CHUNK, _CHUNK)],
        out_hbm.at[pl.ds(base + c * _CHUNK, _CHUNK)],
        sems[c],
    ))
  for w in writes:
    w.wait()


def _sc_gather(node_idx, node_table):
  mesh = plsc.VectorSubcoreMesh(core_axis_name="c", subcore_axis_name="s")
  run = pl.kernel(
      _sc_gather_body,
      out_type=jax.ShapeDtypeStruct((_B_PAD, NODE_D), jnp.float32),
      mesh=mesh,
      scratch_types=[
          pltpu.VMEM((_BPW,), jnp.int32),
          pltpu.VMEM((_BPW, NODE_D), jnp.float32),
      ] + [pltpu.SemaphoreType.DMA] * (_BPW // _CHUNK),
  )
  return run(node_idx, node_table)


def _gcn_body(adj_ref, node_e, idx_ref,
              lane_t, type_t, len_t, wl, wt, wn_, wd, w_ref, b_ref,
              out_ref, s1_ref, s2_ref):
  l = pl.program_id(0)
  i = pl.program_id(1)

  @pl.when((l == 0) & (i == 0))
  def _():
    def onehot(idx_col, kk):
      cols = lax.broadcasted_iota(jnp.int32, (1, kk), 1)
      return (idx_col == cols).astype(jnp.float32)

    lane_w = _DOT(lane_t[...], wl[...])
    type_w = _DOT(type_t[...], wt[...])
    len_w = _DOT(len_t[...], wn_[...])
    for c in range(N // 1000):
      r = pl.ds(c * 1000, 1000)
      s1 = _DOT(node_e[r, :], wd[...])
      s1 += _DOT(onehot(idx_ref[r, 0:1], 16), lane_w)
      s1 += _DOT(onehot(idx_ref[r, 1:2], 24), type_w)
      s1 += _DOT(onehot(idx_ref[r, 2:3], 104), len_w)
      s1_ref[r, :] = s1

  @pl.when(l == 0)
  def _():
    y = jnp.maximum(_DOT(adj_ref[...], s1_ref[...]) + b_ref[...], 0.0)
    s2_ref[pl.ds(i * _BM, _BM), :] = _DOT(y, w_ref[...])

  @pl.when(l == 1)
  def _():
    out_ref[...] = jnp.maximum(
        _DOT(adj_ref[...], s2_ref[...]) + b_ref[...], 0.0)


def _gcn(adj, node_e, idx3, lane_t, type_t, len_t, W, b2d):
  grid = (2, N // _BM)

  const = lambda l, i: (0, 0)
  return pl.pallas_call(
      _gcn_body,
      grid=grid,
      in_specs=[
          pl.BlockSpec((_BM, N), lambda l, i: (i, 0)),
          pl.BlockSpec((N, NODE_D), const),
          pl.BlockSpec((N, 3), const),
          pl.BlockSpec((16, 64), const),
          pl.BlockSpec((24, 32), const),
          pl.BlockSpec((104, 32), const),
          pl.BlockSpec((64, H), const),
          pl.BlockSpec((32, H), const),
          pl.BlockSpec((32, H), const),
          pl.BlockSpec((NODE_D, H), const),
          pl.BlockSpec((H, H), const),
          pl.BlockSpec((1, H), const),
      ],
      out_specs=pl.BlockSpec((_BM, H),
                             lambda l, i: (jnp.where(l == 0, 0, i), 0)),
      out_shape=jax.ShapeDtypeStruct((N, H), jnp.float32),
      scratch_shapes=[
          pltpu.VMEM((N, H), jnp.float32),
          pltpu.VMEM((N, H), jnp.float32),
      ],
      compiler_params=pltpu.CompilerParams(
          dimension_semantics=("arbitrary", "arbitrary")),
  )(adj, node_e, idx3, lane_t, type_t, len_t,
    W[:64], W[64:96], W[96:128], W[128:], W, b2d)


def kernel(node_feature, type_feature, length_feature, lane_feature, adj,
           node_table, type_table, length_table, lane_table, W, b):
  node_idx = jnp.pad(node_feature.astype(jnp.int32), (0, _B_PAD - N))
  node_e = _sc_gather(node_idx, node_table)[:N]

  def padt(t, k_pad):
    return jnp.pad(t, ((0, k_pad - t.shape[0]), (0, 0)))

  idx3 = jnp.stack([lane_feature, type_feature, length_feature],
                   axis=1).astype(jnp.int32)
  return _gcn(adj, node_e, idx3, padt(lane_table, 16),
              padt(type_table, 24), padt(length_table, 104),
              W, b.reshape(1, H))


# speedup vs baseline: 1.0102x; 1.0102x over previous
import functools

import jax
import jax.numpy as jnp
from jax import lax
from jax.experimental import pallas as pl
from jax.experimental.pallas import tpu as pltpu
from jax.experimental.pallas import tpu_sc as plsc

N = 10000
H = 256
NODE_D = 128

_NC, _NS = 2, 16
_NW = _NC * _NS
_B_PAD = 10240
_BPW = _B_PAD // _NW
_CHUNK = 80

_BM = 200

_DOT = functools.partial(jnp.dot, precision=lax.Precision.DEFAULT,
                         preferred_element_type=jnp.float32)


def _sc_gather_body(idx_hbm, table_hbm, out_hbm, idx_v, rows_v, sem):
  wid = lax.axis_index("s") * _NC + lax.axis_index("c")
  base = wid * _BPW
  pltpu.sync_copy(idx_hbm.at[pl.ds(base, _BPW)], idx_v)
  handles = [
      pltpu.async_copy(
          table_hbm.at[idx_v.at[pl.ds(c * _CHUNK, _CHUNK)]],
          rows_v.at[pl.ds(c * _CHUNK, _CHUNK)],
          sem,
      )
      for c in range(_BPW // _CHUNK)
  ]
  for h in handles:
    h.wait()
  pltpu.sync_copy(rows_v, out_hbm.at[pl.ds(base, _BPW)])


def _sc_gather(node_idx, node_table):
  mesh = plsc.VectorSubcoreMesh(core_axis_name="c", subcore_axis_name="s")
  run = pl.kernel(
      _sc_gather_body,
      out_type=jax.ShapeDtypeStruct((_B_PAD, NODE_D), jnp.float32),
      mesh=mesh,
      scratch_types=[
          pltpu.VMEM((_BPW,), jnp.int32),
          pltpu.VMEM((_BPW, NODE_D), jnp.float32),
          pltpu.SemaphoreType.DMA,
      ],
  )
  return run(node_idx, node_table)


def _gcn_body(adj_ref, node_e, idx_ref,
              lane_t, type_t, len_t, wl, wt, wn_, wd, w_ref, b_ref,
              out_ref, s1_ref, s2_ref):
  l = pl.program_id(0)
  i = pl.program_id(1)

  @pl.when((l == 0) & (i == 0))
  def _():
    def onehot(idx_col, kk):
      cols = lax.broadcasted_iota(jnp.int32, (1, kk), 1)
      return (idx_col == cols).astype(jnp.float32)

    lane_w = _DOT(lane_t[...], wl[...])
    type_w = _DOT(type_t[...], wt[...])
    len_w = _DOT(len_t[...], wn_[...])
    for c in range(N // 1000):
      r = pl.ds(c * 1000, 1000)
      s1 = _DOT(node_e[r, :], wd[...])
      s1 += _DOT(onehot(idx_ref[r, 0:1], 16), lane_w)
      s1 += _DOT(onehot(idx_ref[r, 1:2], 24), type_w)
      s1 += _DOT(onehot(idx_ref[r, 2:3], 104), len_w)
      s1_ref[r, :] = s1

  @pl.when(l == 0)
  def _():
    y = jnp.maximum(_DOT(adj_ref[...], s1_ref[...]) + b_ref[...], 0.0)
    s2_ref[pl.ds(i * _BM, _BM), :] = _DOT(y, w_ref[...])

  @pl.when(l == 1)
  def _():
    out_ref[...] = jnp.maximum(
        _DOT(adj_ref[...], s2_ref[...]) + b_ref[...], 0.0)


def _gcn(adj, node_e, idx3, lane_t, type_t, len_t, W, b2d):
  grid = (2, N // _BM)

  const = lambda l, i: (0, 0)
  return pl.pallas_call(
      _gcn_body,
      grid=grid,
      in_specs=[
          pl.BlockSpec((_BM, N), lambda l, i: (i, 0)),
          pl.BlockSpec((N, NODE_D), const),
          pl.BlockSpec((N, 3), const),
          pl.BlockSpec((16, 64), const),
          pl.BlockSpec((24, 32), const),
          pl.BlockSpec((104, 32), const),
          pl.BlockSpec((64, H), const),
          pl.BlockSpec((32, H), const),
          pl.BlockSpec((32, H), const),
          pl.BlockSpec((NODE_D, H), const),
          pl.BlockSpec((H, H), const),
          pl.BlockSpec((1, H), const),
      ],
      out_specs=pl.BlockSpec((_BM, H),
                             lambda l, i: (jnp.where(l == 0, 0, i), 0)),
      out_shape=jax.ShapeDtypeStruct((N, H), jnp.float32),
      scratch_shapes=[
          pltpu.VMEM((N, H), jnp.float32),
          pltpu.VMEM((N, H), jnp.float32),
      ],
      compiler_params=pltpu.CompilerParams(
          dimension_semantics=("arbitrary", "arbitrary")),
  )(adj, node_e, idx3, lane_t, type_t, len_t,
    W[:64], W[64:96], W[96:128], W[128:], W, b2d)


def kernel(node_feature, type_feature, length_feature, lane_feature, adj,
           node_table, type_table, length_table, lane_table, W, b):
  node_idx = jnp.pad(node_feature.astype(jnp.int32), (0, _B_PAD - N))
  node_e = _sc_gather(node_idx, node_table)[:N]

  def padt(t, k_pad):
    return jnp.pad(t, ((0, k_pad - t.shape[0]), (0, 0)))

  idx3 = jnp.stack([lane_feature, type_feature, length_feature],
                   axis=1).astype(jnp.int32)
  return _gcn(adj, node_e, idx3, padt(lane_table, 16),
              padt(type_table, 24), padt(length_table, 104),
              W, b.reshape(1, H))
